# baseline (speedup 1.0000x reference)
import functools

import jax
import jax.numpy as jnp
from jax import lax
from jax.experimental import pallas as pl
from jax.experimental.pallas import tpu as pltpu
from jax.experimental.pallas import tpu_sc as plsc

N = 1024
VOCAB = 100000
DIM = 64

INT_MAX = 2**31 - 1

TBVT = 2048
TC_FULL = 25
TAIL_BLK = 48
TC_STEPS = TC_FULL + 1


def _tct_body(s_ref, a_ref, m_ref):
    j = pl.program_id(0)
    blk = jnp.where(j < TC_FULL, j, TAIL_BLK)
    vidx = lax.broadcasted_iota(jnp.int32, (TBVT, N), 0) + blk * TBVT
    v = jnp.where(vidx < VOCAB, s_ref[...], -jnp.inf)
    m = jnp.max(v, axis=0, keepdims=True)
    a = jnp.min(jnp.where(v == m, vidx, INT_MAX), axis=0, keepdims=True)

    @pl.when(j == 0)
    def _():
        m_ref[...] = m
        a_ref[...] = a

    @pl.when(j > 0)
    def _():
        better = m > m_ref[...]
        m_ref[...] = jnp.where(better, m, m_ref[...])
        a_ref[...] = jnp.where(better, a, a_ref[...])


_tct_scan = pl.pallas_call(
    _tct_body,
    grid=(TC_STEPS,),
    in_specs=[pl.BlockSpec((TBVT, N),
                           lambda j: (jnp.where(j < TC_FULL, j, TAIL_BLK), 0))],
    out_specs=(pl.BlockSpec((1, N), lambda j: (0, 0)),
               pl.BlockSpec((1, N), lambda j: (0, 0))),
    out_shape=(jax.ShapeDtypeStruct((1, N), jnp.int32),
               jax.ShapeDtypeStruct((1, N), jnp.float32)),
    compiler_params=pltpu.CompilerParams(
        dimension_semantics=("arbitrary",),
    ),
)

NC, NS = 2, 16
NW = NC * NS
SC_LO = TC_FULL * TBVT
WS = 1472
CR = 32
NCH = WS // CR
NPAIR = NCH // 2


@functools.cache
def _make_scan_sc():
    mesh = plsc.VectorSubcoreMesh(core_axis_name="c", subcore_axis_name="s")

    @functools.partial(
        pl.kernel,
        mesh=mesh,
        out_type=(jax.ShapeDtypeStruct((NW * N,), jnp.float32),
                  jax.ShapeDtypeStruct((NW * N,), jnp.int32)),
        scratch_types=[
            pltpu.VMEM((CR, N), jnp.float32),
            pltpu.VMEM((CR, N), jnp.float32),
            pltpu.VMEM((N,), jnp.float32),
            pltpu.VMEM((N,), jnp.int32),
            pltpu.SemaphoreType.DMA,
            pltpu.SemaphoreType.DMA,
        ],
        compiler_params=pltpu.CompilerParams(use_tc_tiling_on_sc=True),
    )
    def _scan(st_hbm, mout_hbm, aout_hbm,
              buf_a, buf_b, m_st, a_st, sem_a, sem_b):
        wid = lax.axis_index("s") * NC + lax.axis_index("c")
        stripe0 = SC_LO + wid * WS
        neg_inf = jnp.full((16,), -jnp.inf, jnp.float32)
        zero16 = jnp.zeros((16,), jnp.int32)

        def init_body(i, c):
            m_st[pl.ds(i * 16, 16)] = neg_inf
            a_st[pl.ds(i * 16, 16)] = zero16
            return c
        lax.fori_loop(0, N // 16, init_body, 0)

        def src(c):
            return st_hbm.at[pl.ds(stripe0 + c * CR, CR)]

        def process(c, buf):
            for cg4 in range(N // 64):
                carry0 = (tuple(m_st[pl.ds((cg4 * 4 + k) * 16, 16)]
                                for k in range(4))
                          + tuple(a_st[pl.ds((cg4 * 4 + k) * 16, 16)]
                                  for k in range(4)))

                def body(row, cr, cg4=cg4, buf=buf, c=c):
                    ms = list(cr[:4])
                    as_ = list(cr[4:])
                    vid = jnp.broadcast_to(stripe0 + c * CR + row, (16,))
                    for k in range(4):
                        v = buf[row, pl.ds((cg4 * 4 + k) * 16, 16)]
                        upd = v > ms[k]
                        ms[k] = jnp.where(upd, v, ms[k])
                        as_[k] = jnp.where(upd, vid, as_[k])
                    return tuple(ms) + tuple(as_)

                fin = plsc.parallel_loop(0, CR, carry=carry0)(body)
                for k in range(4):
                    m_st[pl.ds((cg4 * 4 + k) * 16, 16)] = fin[k]
                    a_st[pl.ds((cg4 * 4 + k) * 16, 16)] = fin[4 + k]

        pltpu.async_copy(src(0), buf_a, sem_a)

        def pair_body(p, c):
            c0 = 2 * p
            cp_b = pltpu.async_copy(src(c0 + 1), buf_b, sem_b)
            pltpu.make_async_copy(src(c0), buf_a, sem_a).wait()
            process(c0, buf_a)

            @pl.when(p < NPAIR - 1)
            def _():
                pltpu.async_copy(src(c0 + 2), buf_a, sem_a)

            cp_b.wait()
            process(c0 + 1, buf_b)
            return c
        lax.fori_loop(0, NPAIR, pair_body, 0)

        pltpu.sync_copy(m_st, mout_hbm.at[pl.ds(wid * N, N)])
        pltpu.sync_copy(a_st, aout_hbm.at[pl.ds(wid * N, N)])

    return _scan


def _merge_body(at_ref, mt_ref, mp_ref, ap_ref, best_ref):
    a_t = at_ref[...]
    m_t = mt_ref[...]
    mp = mp_ref[...]
    ap = ap_ref[...]
    m_s = jnp.max(mp, axis=0, keepdims=True)
    a_s = jnp.min(jnp.where(mp == m_s, ap, INT_MAX), axis=0, keepdims=True)
    take = (m_s > m_t) | ((m_s == m_t) & (a_s < a_t))
    best_ref[...] = jnp.where(take, a_s, a_t)


_merge_call = pl.pallas_call(
    _merge_body,
    out_shape=jax.ShapeDtypeStruct((1, N), jnp.int32),
)


GG = 8


def _gather_body(best_sm, *refs):
    ts, out_ref = refs[:GG], refs[GG]
    i = pl.program_id(0)
    lane = lax.broadcasted_iota(jnp.int32, (1, 128), 1)
    rows = []
    for k, t in enumerate(ts):
        c = best_sm[GG * i + k]
        onehot = (lane == c % 128).astype(jnp.float32)
        rows.append(lax.dot_general(onehot, t[...],
                                    (((1,), (1,)), ((), ())),
                                    preferred_element_type=jnp.float32))
    out_ref[...] = jnp.concatenate(rows, axis=0)


def _mk_table_spec(k):
    return pl.BlockSpec((DIM, 128), lambda i, b, k=k: (0, b[GG * i + k] // 128))


_gather_tc = pl.pallas_call(
    _gather_body,
    grid_spec=pltpu.PrefetchScalarGridSpec(
        num_scalar_prefetch=1,
        grid=(N // GG,),
        in_specs=[_mk_table_spec(k) for k in range(GG)],
        out_specs=pl.BlockSpec((GG, DIM), lambda i, b: (i, 0)),
    ),
    out_shape=jax.ShapeDtypeStruct((N, DIM), jnp.float32),
)


def kernel(scores, emb_weight):
    st = jnp.swapaxes(scores, 0, 1)
    a_t, m_t = _tct_scan(st)
    m_flat, a_flat = _make_scan_sc()(st)
    mp = m_flat.reshape(NW, N)
    ap = a_flat.reshape(NW, N)
    best = _merge_call(a_t, m_t, mp, ap).reshape(N)
    tt = jnp.swapaxes(emb_weight, 0, 1)
    emb = _gather_tc(best, *([tt] * GG))
    return emb, best


# speedup vs baseline: 1.0261x; 1.0261x over previous
import functools

import jax
import jax.numpy as jnp
from jax import lax
from jax.experimental import pallas as pl
from jax.experimental.pallas import tpu as pltpu
from jax.experimental.pallas import tpu_sc as plsc

N = 1024
VOCAB = 100000
DIM = 64

INT_MAX = 2**31 - 1

TBVT = 2048
TC_FULL = 25
TAIL_BLK = 48
TC_STEPS = TC_FULL + 1


def _tct_body(s_ref, a_ref, m_ref):
    j = pl.program_id(0)
    blk = jnp.where(j < TC_FULL, j, TAIL_BLK)
    vidx = lax.broadcasted_iota(jnp.int32, (TBVT, N), 0) + blk * TBVT
    v = jnp.where(vidx < VOCAB, s_ref[...], -jnp.inf)
    m = jnp.max(v, axis=0, keepdims=True)
    a = jnp.min(jnp.where(v == m, vidx, INT_MAX), axis=0, keepdims=True)

    @pl.when(j == 0)
    def _():
        m_ref[...] = m
        a_ref[...] = a

    @pl.when(j > 0)
    def _():
        better = m > m_ref[...]
        m_ref[...] = jnp.where(better, m, m_ref[...])
        a_ref[...] = jnp.where(better, a, a_ref[...])


_tct_scan = pl.pallas_call(
    _tct_body,
    grid=(TC_STEPS,),
    in_specs=[pl.BlockSpec((TBVT, N),
                           lambda j: (jnp.where(j < TC_FULL, j, TAIL_BLK), 0))],
    out_specs=(pl.BlockSpec((1, N), lambda j: (0, 0)),
               pl.BlockSpec((1, N), lambda j: (0, 0))),
    out_shape=(jax.ShapeDtypeStruct((1, N), jnp.int32),
               jax.ShapeDtypeStruct((1, N), jnp.float32)),
    compiler_params=pltpu.CompilerParams(
        dimension_semantics=("arbitrary",),
    ),
)

NC, NS = 2, 16
NW = NC * NS
SC_LO = TC_FULL * TBVT
WS = 1472
CR = 32
NCH = WS // CR
NPAIR = NCH // 2


@functools.cache
def _make_scan_sc():
    mesh = plsc.VectorSubcoreMesh(core_axis_name="c", subcore_axis_name="s")

    @functools.partial(
        pl.kernel,
        mesh=mesh,
        out_type=(jax.ShapeDtypeStruct((NW * N,), jnp.float32),
                  jax.ShapeDtypeStruct((NW * N,), jnp.int32)),
        scratch_types=[
            pltpu.VMEM((CR, N), jnp.float32),
            pltpu.VMEM((CR, N), jnp.float32),
            pltpu.VMEM((N,), jnp.float32),
            pltpu.VMEM((N,), jnp.int32),
            pltpu.SemaphoreType.DMA,
            pltpu.SemaphoreType.DMA,
        ],
        compiler_params=pltpu.CompilerParams(use_tc_tiling_on_sc=True),
    )
    def _scan(st_hbm, mout_hbm, aout_hbm,
              buf_a, buf_b, m_st, a_st, sem_a, sem_b):
        wid = lax.axis_index("s") * NC + lax.axis_index("c")
        stripe0 = SC_LO + wid * WS
        neg_inf = jnp.full((16,), -jnp.inf, jnp.float32)
        zero16 = jnp.zeros((16,), jnp.int32)

        def init_body(i, c):
            m_st[pl.ds(i * 16, 16)] = neg_inf
            a_st[pl.ds(i * 16, 16)] = zero16
            return c
        lax.fori_loop(0, N // 16, init_body, 0)

        def src(c):
            return st_hbm.at[pl.ds(stripe0 + c * CR, CR)]

        def process(c, buf):
            for cg4 in range(N // 64):
                carry0 = (tuple(m_st[pl.ds((cg4 * 4 + k) * 16, 16)]
                                for k in range(4))
                          + tuple(a_st[pl.ds((cg4 * 4 + k) * 16, 16)]
                                  for k in range(4)))

                def body(row, cr, cg4=cg4, buf=buf, c=c):
                    ms = list(cr[:4])
                    as_ = list(cr[4:])
                    vid = jnp.broadcast_to(stripe0 + c * CR + row, (16,))
                    for k in range(4):
                        v = buf[row, pl.ds((cg4 * 4 + k) * 16, 16)]
                        upd = v > ms[k]
                        ms[k] = jnp.where(upd, v, ms[k])
                        as_[k] = jnp.where(upd, vid, as_[k])
                    return tuple(ms) + tuple(as_)

                fin = plsc.parallel_loop(0, CR, carry=carry0)(body)
                for k in range(4):
                    m_st[pl.ds((cg4 * 4 + k) * 16, 16)] = fin[k]
                    a_st[pl.ds((cg4 * 4 + k) * 16, 16)] = fin[4 + k]

        pltpu.async_copy(src(0), buf_a, sem_a)

        def pair_body(p, c):
            c0 = 2 * p
            cp_b = pltpu.async_copy(src(c0 + 1), buf_b, sem_b)
            pltpu.make_async_copy(src(c0), buf_a, sem_a).wait()
            process(c0, buf_a)

            @pl.when(p < NPAIR - 1)
            def _():
                pltpu.async_copy(src(c0 + 2), buf_a, sem_a)

            cp_b.wait()
            process(c0 + 1, buf_b)
            return c
        lax.fori_loop(0, NPAIR, pair_body, 0)

        pltpu.sync_copy(m_st, mout_hbm.at[pl.ds(wid * N, N)])
        pltpu.sync_copy(a_st, aout_hbm.at[pl.ds(wid * N, N)])

    return _scan


def _merge_body(at_ref, mt_ref, mp_ref, ap_ref, best_ref):
    a_t = at_ref[...]
    m_t = mt_ref[...]
    mp = mp_ref[...]
    ap = ap_ref[...]
    m_s = jnp.max(mp, axis=0, keepdims=True)
    a_s = jnp.min(jnp.where(mp == m_s, ap, INT_MAX), axis=0, keepdims=True)
    take = (m_s > m_t) | ((m_s == m_t) & (a_s < a_t))
    best_ref[...] = jnp.where(take, a_s, a_t)


_merge_call = pl.pallas_call(
    _merge_body,
    out_shape=jax.ShapeDtypeStruct((1, N), jnp.int32),
)


GG = 8


def _gather_body(best_sm, *refs):
    ts, out_ref = refs[:GG], refs[GG]
    i = pl.program_id(0)
    lane = lax.broadcasted_iota(jnp.int32, (GG, 128), 1)
    row = lax.broadcasted_iota(jnp.int32, (GG, 128), 0)
    acc = None
    for k, t in enumerate(ts):
        c = best_sm[GG * i + k]
        oh = ((lane == c % 128) & (row == k)).astype(jnp.float32)
        p = lax.dot_general(oh, t[...], (((1,), (1,)), ((), ())),
                            preferred_element_type=jnp.float32)
        acc = p if acc is None else acc + p
    out_ref[...] = acc


def _mk_table_spec(k):
    return pl.BlockSpec((DIM, 128), lambda i, b, k=k: (0, b[GG * i + k] // 128))


_gather_tc = pl.pallas_call(
    _gather_body,
    grid_spec=pltpu.PrefetchScalarGridSpec(
        num_scalar_prefetch=1,
        grid=(N // GG,),
        in_specs=[_mk_table_spec(k) for k in range(GG)],
        out_specs=pl.BlockSpec((GG, DIM), lambda i, b: (i, 0)),
    ),
    out_shape=jax.ShapeDtypeStruct((N, DIM), jnp.float32),
)


def kernel(scores, emb_weight):
    st = jnp.swapaxes(scores, 0, 1)
    a_t, m_t = _tct_scan(st)
    m_flat, a_flat = _make_scan_sc()(st)
    mp = m_flat.reshape(NW, N)
    ap = a_flat.reshape(NW, N)
    best = _merge_call(a_t, m_t, mp, ap).reshape(N)
    tt = jnp.swapaxes(emb_weight, 0, 1)
    emb = _gather_tc(best, *([tt] * GG))
    return emb, best


# speedup vs baseline: 1.1576x; 1.1282x over previous
import functools

import jax
import jax.numpy as jnp
from jax import lax
from jax.experimental import pallas as pl
from jax.experimental.pallas import tpu as pltpu
from jax.experimental.pallas import tpu_sc as plsc

N = 1024
VOCAB = 100000
DIM = 64

INT_MAX = 2**31 - 1

TBVT = 2048
TC_FULL = 25
TAIL_BLK = 48
TC_STEPS = TC_FULL + 1


def _tct_body(s_ref, a_ref, m_ref):
    j = pl.program_id(0)
    blk = jnp.where(j < TC_FULL, j, TAIL_BLK)
    vidx = lax.broadcasted_iota(jnp.int32, (TBVT, N), 0) + blk * TBVT
    v = jnp.where(vidx < VOCAB, s_ref[...], -jnp.inf)
    m = jnp.max(v, axis=0, keepdims=True)
    a = jnp.min(jnp.where(v == m, vidx, INT_MAX), axis=0, keepdims=True)

    @pl.when(j == 0)
    def _():
        m_ref[...] = m
        a_ref[...] = a

    @pl.when(j > 0)
    def _():
        better = m > m_ref[...]
        m_ref[...] = jnp.where(better, m, m_ref[...])
        a_ref[...] = jnp.where(better, a, a_ref[...])


_tct_scan = pl.pallas_call(
    _tct_body,
    grid=(TC_STEPS,),
    in_specs=[pl.BlockSpec((TBVT, N),
                           lambda j: (jnp.where(j < TC_FULL, j, TAIL_BLK), 0))],
    out_specs=(pl.BlockSpec((1, N), lambda j: (0, 0)),
               pl.BlockSpec((1, N), lambda j: (0, 0))),
    out_shape=(jax.ShapeDtypeStruct((1, N), jnp.int32),
               jax.ShapeDtypeStruct((1, N), jnp.float32)),
    compiler_params=pltpu.CompilerParams(
        dimension_semantics=("arbitrary",),
    ),
)

NC, NS = 2, 16
NW = NC * NS
SC_LO = TC_FULL * TBVT
WS = 1472
CR = 32
NCH = WS // CR
NPAIR = NCH // 2


@functools.cache
def _make_scan_sc():
    mesh = plsc.VectorSubcoreMesh(core_axis_name="c", subcore_axis_name="s")

    @functools.partial(
        pl.kernel,
        mesh=mesh,
        out_type=(jax.ShapeDtypeStruct((NW * N,), jnp.float32),
                  jax.ShapeDtypeStruct((NW * N,), jnp.int32)),
        scratch_types=[
            pltpu.VMEM((CR, N), jnp.float32),
            pltpu.VMEM((CR, N), jnp.float32),
            pltpu.VMEM((N,), jnp.float32),
            pltpu.VMEM((N,), jnp.int32),
            pltpu.SemaphoreType.DMA,
            pltpu.SemaphoreType.DMA,
        ],
        compiler_params=pltpu.CompilerParams(use_tc_tiling_on_sc=True),
    )
    def _scan(st_hbm, mout_hbm, aout_hbm,
              buf_a, buf_b, m_st, a_st, sem_a, sem_b):
        wid = lax.axis_index("s") * NC + lax.axis_index("c")
        stripe0 = SC_LO + wid * WS
        neg_inf = jnp.full((16,), -jnp.inf, jnp.float32)
        zero16 = jnp.zeros((16,), jnp.int32)

        def init_body(i, c):
            m_st[pl.ds(i * 16, 16)] = neg_inf
            a_st[pl.ds(i * 16, 16)] = zero16
            return c
        lax.fori_loop(0, N // 16, init_body, 0)

        def src(c):
            return st_hbm.at[pl.ds(stripe0 + c * CR, CR)]

        def process(c, buf):
            for cg4 in range(N // 64):
                carry0 = (tuple(m_st[pl.ds((cg4 * 4 + k) * 16, 16)]
                                for k in range(4))
                          + tuple(a_st[pl.ds((cg4 * 4 + k) * 16, 16)]
                                  for k in range(4)))

                def body(row, cr, cg4=cg4, buf=buf, c=c):
                    ms = list(cr[:4])
                    as_ = list(cr[4:])
                    vid = jnp.broadcast_to(stripe0 + c * CR + row, (16,))
                    for k in range(4):
                        v = buf[row, pl.ds((cg4 * 4 + k) * 16, 16)]
                        upd = v > ms[k]
                        ms[k] = jnp.where(upd, v, ms[k])
                        as_[k] = jnp.where(upd, vid, as_[k])
                    return tuple(ms) + tuple(as_)

                fin = plsc.parallel_loop(0, CR, carry=carry0)(body)
                for k in range(4):
                    m_st[pl.ds((cg4 * 4 + k) * 16, 16)] = fin[k]
                    a_st[pl.ds((cg4 * 4 + k) * 16, 16)] = fin[4 + k]

        pltpu.async_copy(src(0), buf_a, sem_a)

        def pair_body(p, c):
            c0 = 2 * p
            cp_b = pltpu.async_copy(src(c0 + 1), buf_b, sem_b)
            pltpu.make_async_copy(src(c0), buf_a, sem_a).wait()
            process(c0, buf_a)

            @pl.when(p < NPAIR - 1)
            def _():
                pltpu.async_copy(src(c0 + 2), buf_a, sem_a)

            cp_b.wait()
            process(c0 + 1, buf_b)
            return c
        lax.fori_loop(0, NPAIR, pair_body, 0)

        pltpu.sync_copy(m_st, mout_hbm.at[pl.ds(wid * N, N)])
        pltpu.sync_copy(a_st, aout_hbm.at[pl.ds(wid * N, N)])

    return _scan


def _merge_body(at_ref, mt_ref, mp_ref, ap_ref, best_ref):
    a_t = at_ref[...]
    m_t = mt_ref[...]
    mp = mp_ref[...]
    ap = ap_ref[...]
    m_s = jnp.max(mp, axis=0, keepdims=True)
    a_s = jnp.min(jnp.where(mp == m_s, ap, INT_MAX), axis=0, keepdims=True)
    take = (m_s > m_t) | ((m_s == m_t) & (a_s < a_t))
    best_ref[...] = jnp.where(take, a_s, a_t)


_merge_call = pl.pallas_call(
    _merge_body,
    out_shape=jax.ShapeDtypeStruct((1, N), jnp.int32),
)


GG = 16


def _gather_body(best_sm, *refs):
    ts, out_ref = refs[:GG], refs[GG]
    i = pl.program_id(0)
    lane = lax.broadcasted_iota(jnp.int32, (GG, 128), 1)
    row = lax.broadcasted_iota(jnp.int32, (GG, 128), 0)
    acc = None
    for k, t in enumerate(ts):
        c = best_sm[GG * i + k]
        oh = ((lane == c % 128) & (row == k)).astype(jnp.float32)
        p = lax.dot_general(oh, t[...], (((1,), (1,)), ((), ())),
                            preferred_element_type=jnp.float32)
        acc = p if acc is None else acc + p
    out_ref[...] = acc


def _mk_table_spec(k):
    return pl.BlockSpec((DIM, 128), lambda i, b, k=k: (0, b[GG * i + k] // 128))


_gather_tc = pl.pallas_call(
    _gather_body,
    grid_spec=pltpu.PrefetchScalarGridSpec(
        num_scalar_prefetch=1,
        grid=(N // GG,),
        in_specs=[_mk_table_spec(k) for k in range(GG)],
        out_specs=pl.BlockSpec((GG, DIM), lambda i, b: (i, 0)),
    ),
    out_shape=jax.ShapeDtypeStruct((N, DIM), jnp.float32),
)


def kernel(scores, emb_weight):
    st = jnp.swapaxes(scores, 0, 1)
    a_t, m_t = _tct_scan(st)
    m_flat, a_flat = _make_scan_sc()(st)
    mp = m_flat.reshape(NW, N)
    ap = a_flat.reshape(NW, N)
    best = _merge_call(a_t, m_t, mp, ap).reshape(N)
    tt = jnp.swapaxes(emb_weight, 0, 1)
    emb = _gather_tc(best, *([tt] * GG))
    return emb, best


# speedup vs baseline: 1.2073x; 1.0429x over previous
import functools

import jax
import jax.numpy as jnp
from jax import lax
from jax.experimental import pallas as pl
from jax.experimental.pallas import tpu as pltpu
from jax.experimental.pallas import tpu_sc as plsc

N = 1024
VOCAB = 100000
DIM = 64

INT_MAX = 2**31 - 1

TBVT = 2048
TC_FULL = 25
TAIL_BLK = 48
TC_STEPS = TC_FULL + 1


def _tct_body(s_ref, a_ref, m_ref):
    j = pl.program_id(0)
    blk = jnp.where(j < TC_FULL, j, TAIL_BLK)
    vidx = lax.broadcasted_iota(jnp.int32, (TBVT, N), 0) + blk * TBVT
    v = jnp.where(vidx < VOCAB, s_ref[...], -jnp.inf)
    m = jnp.max(v, axis=0, keepdims=True)
    a = jnp.min(jnp.where(v == m, vidx, INT_MAX), axis=0, keepdims=True)

    @pl.when(j == 0)
    def _():
        m_ref[...] = m
        a_ref[...] = a

    @pl.when(j > 0)
    def _():
        better = m > m_ref[...]
        m_ref[...] = jnp.where(better, m, m_ref[...])
        a_ref[...] = jnp.where(better, a, a_ref[...])


_tct_scan = pl.pallas_call(
    _tct_body,
    grid=(TC_STEPS,),
    in_specs=[pl.BlockSpec((TBVT, N),
                           lambda j: (jnp.where(j < TC_FULL, j, TAIL_BLK), 0))],
    out_specs=(pl.BlockSpec((1, N), lambda j: (0, 0)),
               pl.BlockSpec((1, N), lambda j: (0, 0))),
    out_shape=(jax.ShapeDtypeStruct((1, N), jnp.int32),
               jax.ShapeDtypeStruct((1, N), jnp.float32)),
    compiler_params=pltpu.CompilerParams(
        dimension_semantics=("arbitrary",),
    ),
)

NC, NS = 2, 16
NW = NC * NS
SC_LO = TC_FULL * TBVT
WS = 1472
CR = 32
NCH = WS // CR
NPAIR = NCH // 2


@functools.cache
def _make_scan_sc():
    mesh = plsc.VectorSubcoreMesh(core_axis_name="c", subcore_axis_name="s")

    @functools.partial(
        pl.kernel,
        mesh=mesh,
        out_type=(jax.ShapeDtypeStruct((NW * N,), jnp.float32),
                  jax.ShapeDtypeStruct((NW * N,), jnp.int32)),
        scratch_types=[
            pltpu.VMEM((CR, N), jnp.float32),
            pltpu.VMEM((CR, N), jnp.float32),
            pltpu.VMEM((N,), jnp.float32),
            pltpu.VMEM((N,), jnp.int32),
            pltpu.SemaphoreType.DMA,
            pltpu.SemaphoreType.DMA,
        ],
        compiler_params=pltpu.CompilerParams(use_tc_tiling_on_sc=True),
    )
    def _scan(st_hbm, mout_hbm, aout_hbm,
              buf_a, buf_b, m_st, a_st, sem_a, sem_b):
        wid = lax.axis_index("s") * NC + lax.axis_index("c")
        stripe0 = SC_LO + wid * WS
        neg_inf = jnp.full((16,), -jnp.inf, jnp.float32)
        zero16 = jnp.zeros((16,), jnp.int32)

        def init_body(i, c):
            m_st[pl.ds(i * 16, 16)] = neg_inf
            a_st[pl.ds(i * 16, 16)] = zero16
            return c
        lax.fori_loop(0, N // 16, init_body, 0)

        def src(c):
            return st_hbm.at[pl.ds(stripe0 + c * CR, CR)]

        def process(c, buf):
            for cg4 in range(N // 64):
                carry0 = (tuple(m_st[pl.ds((cg4 * 4 + k) * 16, 16)]
                                for k in range(4))
                          + tuple(a_st[pl.ds((cg4 * 4 + k) * 16, 16)]
                                  for k in range(4)))

                def body(row, cr, cg4=cg4, buf=buf, c=c):
                    ms = list(cr[:4])
                    as_ = list(cr[4:])
                    vid = jnp.broadcast_to(stripe0 + c * CR + row, (16,))
                    for k in range(4):
                        v = buf[row, pl.ds((cg4 * 4 + k) * 16, 16)]
                        upd = v > ms[k]
                        ms[k] = jnp.where(upd, v, ms[k])
                        as_[k] = jnp.where(upd, vid, as_[k])
                    return tuple(ms) + tuple(as_)

                fin = plsc.parallel_loop(0, CR, carry=carry0)(body)
                for k in range(4):
                    m_st[pl.ds((cg4 * 4 + k) * 16, 16)] = fin[k]
                    a_st[pl.ds((cg4 * 4 + k) * 16, 16)] = fin[4 + k]

        pltpu.async_copy(src(0), buf_a, sem_a)

        def pair_body(p, c):
            c0 = 2 * p
            cp_b = pltpu.async_copy(src(c0 + 1), buf_b, sem_b)
            pltpu.make_async_copy(src(c0), buf_a, sem_a).wait()
            process(c0, buf_a)

            @pl.when(p < NPAIR - 1)
            def _():
                pltpu.async_copy(src(c0 + 2), buf_a, sem_a)

            cp_b.wait()
            process(c0 + 1, buf_b)
            return c
        lax.fori_loop(0, NPAIR, pair_body, 0)

        pltpu.sync_copy(m_st, mout_hbm.at[pl.ds(wid * N, N)])
        pltpu.sync_copy(a_st, aout_hbm.at[pl.ds(wid * N, N)])

    return _scan


def _merge_body(at_ref, mt_ref, mp_ref, ap_ref, best_ref):
    a_t = at_ref[...]
    m_t = mt_ref[...]
    mp = mp_ref[...]
    ap = ap_ref[...]
    m_s = jnp.max(mp, axis=0, keepdims=True)
    a_s = jnp.min(jnp.where(mp == m_s, ap, INT_MAX), axis=0, keepdims=True)
    take = (m_s > m_t) | ((m_s == m_t) & (a_s < a_t))
    best_ref[...] = jnp.where(take, a_s, a_t)


_merge_call = pl.pallas_call(
    _merge_body,
    out_shape=jax.ShapeDtypeStruct((1, N), jnp.int32),
)


GG = 32


def _gather_body(best_sm, *refs):
    ts, out_ref = refs[:GG], refs[GG]
    i = pl.program_id(0)
    lane = lax.broadcasted_iota(jnp.int32, (GG, 128), 1)
    row = lax.broadcasted_iota(jnp.int32, (GG, 128), 0)
    acc = None
    for k, t in enumerate(ts):
        c = best_sm[GG * i + k]
        oh = ((lane == c % 128) & (row == k)).astype(jnp.float32)
        p = lax.dot_general(oh, t[...], (((1,), (1,)), ((), ())),
                            preferred_element_type=jnp.float32)
        acc = p if acc is None else acc + p
    out_ref[...] = acc


def _mk_table_spec(k):
    return pl.BlockSpec((DIM, 128), lambda i, b, k=k: (0, b[GG * i + k] // 128))


_gather_tc = pl.pallas_call(
    _gather_body,
    grid_spec=pltpu.PrefetchScalarGridSpec(
        num_scalar_prefetch=1,
        grid=(N // GG,),
        in_specs=[_mk_table_spec(k) for k in range(GG)],
        out_specs=pl.BlockSpec((GG, DIM), lambda i, b: (i, 0)),
    ),
    out_shape=jax.ShapeDtypeStruct((N, DIM), jnp.float32),
)


def kernel(scores, emb_weight):
    st = jnp.swapaxes(scores, 0, 1)
    a_t, m_t = _tct_scan(st)
    m_flat, a_flat = _make_scan_sc()(st)
    mp = m_flat.reshape(NW, N)
    ap = a_flat.reshape(NW, N)
    best = _merge_call(a_t, m_t, mp, ap).reshape(N)
    tt = jnp.swapaxes(emb_weight, 0, 1)
    emb = _gather_tc(best, *([tt] * GG))
    return emb, best
